# baseline (speedup 1.0000x reference)
import functools

import jax
import jax.numpy as jnp
from jax import lax
from jax.experimental import pallas as pl
from jax.experimental.pallas import tpu as pltpu
from jax.experimental.pallas import tpu_sc as plsc

_NC = 2
_NS = 16
_NW = _NC * _NS
_L = 16

_W = 4000
_K = 2000


def _build(C: int, n_in: int, n_out: int):
    assert n_out % _W == 0 and n_in % _K == 0
    n_win = n_out // _W
    win_per_worker = -(-n_win // _NW)
    n_chunk = n_in // _K
    chunk_per_tile = -(-n_chunk // _NS)

    mesh = plsc.VectorSubcoreMesh(core_axis_name="c", subcore_axis_name="s")

    @functools.partial(
        pl.kernel,
        mesh=mesh,
        out_type=(
            jax.ShapeDtypeStruct((C, n_out), jnp.float32),
            jax.ShapeDtypeStruct((_NC, n_in, C), jnp.float32),
        ),
        scratch_types=[
            pltpu.VMEM((_W,), jnp.int32),
            pltpu.VMEM((_W, C), jnp.float32),
            pltpu.VMEM((C, _W), jnp.float32),
            pltpu.VMEM((C, _K), jnp.float32),
            pltpu.VMEM((_K, C), jnp.float32),
            pltpu.SemaphoreType.DMA,
        ],
        compiler_params=pltpu.CompilerParams(
            use_tc_tiling_on_sc=False, needs_layout_passes=False
        ),
    )
    def gather_kernel(
        feat_hbm, idx_hbm, out_hbm, tbl_hbm,
        idx_v, rows_v, trans_v, slab_v, rowb_v, sem,
    ):
        core = lax.axis_index("c")
        sub = lax.axis_index("s")
        wid = sub * _NC + core
        lane = lax.iota(jnp.int32, _L)
        c_splat = [jnp.full((_L,), c, dtype=jnp.int32) for c in range(C)]

        def chunk(t, carry):
            k = t * _NS + sub

            @pl.when(k < n_chunk)
            def _():
                base = k * _K
                for c in range(C):
                    pltpu.sync_copy(
                        feat_hbm.at[pl.ds(c * n_in + base, _K)], slab_v.at[c]
                    )

                def grp(g, c2):
                    j = g * _L
                    j_idx = j + lane
                    for c in range(C):
                        vals = slab_v[c, pl.ds(j, _L)]
                        plsc.store_scatter(rowb_v, [j_idx, c_splat[c]], vals)
                    return c2

                lax.fori_loop(0, _K // _L, grp, 0)
                pltpu.sync_copy(rowb_v, tbl_hbm.at[core].at[pl.ds(base, _K), :])

            return carry

        lax.fori_loop(0, chunk_per_tile, chunk, 0)
        plsc.subcore_barrier()

        my_tbl = tbl_hbm.at[core]

        def win(t, carry):
            w = t * _NW + wid

            @pl.when(w < n_win)
            def _():
                base = w * _W
                pltpu.sync_copy(idx_hbm.at[pl.ds(base, _W)], idx_v)
                pltpu.async_copy(my_tbl.at[idx_v], rows_v, sem).wait()

                def grp(g, c2):
                    j = g * _L
                    j_idx = j + lane
                    for c in range(C):
                        vals = plsc.load_gather(rows_v, [j_idx, c_splat[c]])
                        trans_v[c, pl.ds(j, _L)] = vals
                    return c2

                lax.fori_loop(0, _W // _L, grp, 0)
                pltpu.sync_copy(trans_v, out_hbm.at[:, pl.ds(base, _W)])

            return carry

        lax.fori_loop(0, win_per_worker, win, 0)

    return gather_kernel


def kernel(input_features, aprs, level_deltas):
    B, C, n_in = input_features.shape
    n_out = aprs.shape[0]
    feat = input_features.reshape(B * C * n_in)
    out, _ = _build(B * C, n_in, n_out)(feat, aprs)
    return out.reshape(B, C, n_out)


# speedup vs baseline: 1.0375x; 1.0375x over previous
import functools

import jax
import jax.numpy as jnp
from jax import lax
from jax.experimental import pallas as pl
from jax.experimental.pallas import tpu as pltpu
from jax.experimental.pallas import tpu_sc as plsc

_NC = 2
_NS = 16
_NW = _NC * _NS
_L = 16

_W = 6400
_KA = 3200


def _build_transpose(C: int, n_in: int):
    n_full = n_in // _KA
    tail = n_in - n_full * _KA
    n_chunk = n_full + (1 if tail else 0)
    per_tile = -(-n_chunk // _NW)

    mesh = plsc.VectorSubcoreMesh(core_axis_name="c", subcore_axis_name="s")
    scratch = [
        pltpu.VMEM((C, _KA), jnp.float32),
        pltpu.VMEM((_KA * C,), jnp.float32),
    ]
    if tail:
        scratch += [
            pltpu.VMEM((C, tail), jnp.float32),
            pltpu.VMEM((tail * C,), jnp.float32),
        ]
    scratch.append(pltpu.SemaphoreType.DMA)

    @functools.partial(
        pl.kernel,
        mesh=mesh,
        out_type=jax.ShapeDtypeStruct((n_in * C,), jnp.float32),
        scratch_types=scratch,
        compiler_params=pltpu.CompilerParams(needs_layout_passes=False),
    )
    def transpose_kernel(feat_hbm, tbl_hbm, *rest):
        if tail:
            slab_v, rowb_v, slab2_v, rowb2_v, sem = rest
        else:
            slab_v, rowb_v, sem = rest
        core = lax.axis_index("c")
        sub = lax.axis_index("s")
        wid = sub * _NC + core
        lane_c = lax.iota(jnp.int32, _L) * C

        def do(slab, rowb, base, K):
            pltpu.sync_copy(feat_hbm.at[0].at[:, pl.ds(base, K)], slab)

            def grp(g, c2):
                j = g * _L
                for c in range(C):
                    vals = slab[c, pl.ds(j, _L)]
                    plsc.store_scatter(rowb, [j * C + lane_c + c], vals)
                return c2

            lax.fori_loop(0, K // _L, grp, 0)
            pltpu.sync_copy(rowb, tbl_hbm.at[pl.ds(base * C, K * C)])

        def chunk(t, carry):
            m = t * _NW + wid

            @pl.when(m < n_full)
            def _():
                do(slab_v, rowb_v, m * _KA, _KA)

            if tail:
                @pl.when(m == n_full)
                def _():
                    do(slab2_v, rowb2_v, n_full * _KA, tail)

            return carry

        lax.fori_loop(0, per_tile, chunk, 0)

    return transpose_kernel


def _build_gather(C: int, n_in: int, n_out: int):
    assert n_out % _W == 0
    n_win = n_out // _W
    win_per_worker = -(-n_win // _NW)

    mesh = plsc.VectorSubcoreMesh(core_axis_name="c", subcore_axis_name="s")

    @functools.partial(
        pl.kernel,
        mesh=mesh,
        out_type=jax.ShapeDtypeStruct((C, n_out), jnp.float32),
        scratch_types=[
            pltpu.VMEM((_W,), jnp.int32),
            pltpu.VMEM((_W, C), jnp.float32),
            pltpu.VMEM((C, _W), jnp.float32),
            pltpu.SemaphoreType.DMA,
        ],
        compiler_params=pltpu.CompilerParams(
            use_tc_tiling_on_sc=False, needs_layout_passes=False
        ),
    )
    def gather_kernel(tbl_hbm, idx_hbm, out_hbm, idx_v, rows_v, trans_v, sem):
        core = lax.axis_index("c")
        sub = lax.axis_index("s")
        wid = sub * _NC + core
        lane = lax.iota(jnp.int32, _L)
        c_splat = [jnp.full((_L,), c, dtype=jnp.int32) for c in range(C)]

        def win(t, carry):
            w = t * _NW + wid

            @pl.when(w < n_win)
            def _():
                base = w * _W
                pltpu.sync_copy(idx_hbm.at[pl.ds(base, _W)], idx_v)
                pltpu.async_copy(tbl_hbm.at[idx_v], rows_v, sem).wait()

                def grp(g, c2):
                    j = g * _L
                    j_idx = j + lane
                    for c in range(C):
                        vals = plsc.load_gather(rows_v, [j_idx, c_splat[c]])
                        trans_v[c, pl.ds(j, _L)] = vals
                    return c2

                lax.fori_loop(0, _W // _L, grp, 0)
                pltpu.sync_copy(trans_v, out_hbm.at[:, pl.ds(base, _W)])

            return carry

        lax.fori_loop(0, win_per_worker, win, 0)

    return gather_kernel


def kernel(input_features, aprs, level_deltas):
    B, C, n_in = input_features.shape
    n_out = aprs.shape[0]
    tbl_flat = _build_transpose(B * C, n_in)(input_features)
    tbl = tbl_flat.reshape(n_in, B * C)
    out = _build_gather(B * C, n_in, n_out)(tbl, aprs)
    return out.reshape(B, C, n_out)


# speedup vs baseline: 1.0381x; 1.0006x over previous
import functools

import jax
import jax.numpy as jnp
from jax import lax
from jax.experimental import pallas as pl
from jax.experimental.pallas import tpu as pltpu
from jax.experimental.pallas import tpu_sc as plsc

_NC = 2
_NS = 16
_NW = _NC * _NS
_L = 16

_W = 6400
_KA = 3200


def _build_transpose(C: int, n_in: int):
    n_full = n_in // _KA
    tail = n_in - n_full * _KA
    n_chunk = n_full + (1 if tail else 0)
    per_tile = -(-n_chunk // _NW)

    mesh = plsc.VectorSubcoreMesh(core_axis_name="c", subcore_axis_name="s")
    scratch = [
        pltpu.VMEM((C, _KA), jnp.float32),
        pltpu.VMEM((_KA * C,), jnp.float32),
    ]
    if tail:
        scratch += [
            pltpu.VMEM((C, tail), jnp.float32),
            pltpu.VMEM((tail * C,), jnp.float32),
        ]
    scratch.append(pltpu.SemaphoreType.DMA)

    @functools.partial(
        pl.kernel,
        mesh=mesh,
        out_type=jax.ShapeDtypeStruct((n_in * C,), jnp.float32),
        scratch_types=scratch,
        compiler_params=pltpu.CompilerParams(needs_layout_passes=False),
    )
    def transpose_kernel(feat_hbm, tbl_hbm, *rest):
        if tail:
            slab_v, rowb_v, slab2_v, rowb2_v, sem = rest
        else:
            slab_v, rowb_v, sem = rest
        core = lax.axis_index("c")
        sub = lax.axis_index("s")
        wid = sub * _NC + core
        lane_c = lax.iota(jnp.int32, _L) * C

        def do(slab, rowb, base, K):
            pltpu.sync_copy(feat_hbm.at[0].at[:, pl.ds(base, K)], slab)

            def grp(g, c2):
                j = g * _L
                for c in range(C):
                    vals = slab[c, pl.ds(j, _L)]
                    plsc.store_scatter(rowb, [j * C + lane_c + c], vals)
                return c2

            lax.fori_loop(0, K // _L, grp, 0)
            pltpu.sync_copy(rowb, tbl_hbm.at[pl.ds(base * C, K * C)])

        def chunk(t, carry):
            m = t * _NW + wid

            @pl.when(m < n_full)
            def _():
                do(slab_v, rowb_v, m * _KA, _KA)

            if tail:
                @pl.when(m == n_full)
                def _():
                    do(slab2_v, rowb2_v, n_full * _KA, tail)

            return carry

        lax.fori_loop(0, per_tile, chunk, 0)

    return transpose_kernel


def _build_gather(C: int, n_in: int, n_out: int):
    assert n_out % _W == 0
    n_win = n_out // _W
    win_per_worker = -(-n_win // _NW)

    mesh = plsc.VectorSubcoreMesh(core_axis_name="c", subcore_axis_name="s")

    @functools.partial(
        pl.kernel,
        mesh=mesh,
        out_type=jax.ShapeDtypeStruct((1, C, n_out), jnp.float32),
        scratch_types=[
            pltpu.VMEM((_W,), jnp.int32),
            pltpu.VMEM((_W, C), jnp.float32),
            pltpu.VMEM((C, _W), jnp.float32),
            pltpu.SemaphoreType.DMA,
        ],
        compiler_params=pltpu.CompilerParams(
            use_tc_tiling_on_sc=False, needs_layout_passes=False
        ),
    )
    def gather_kernel(tbl_hbm, idx_hbm, out_hbm, idx_v, rows_v, trans_v, sem):
        core = lax.axis_index("c")
        sub = lax.axis_index("s")
        wid = sub * _NC + core
        lane = lax.iota(jnp.int32, _L)
        c_splat = [jnp.full((_L,), c, dtype=jnp.int32) for c in range(C)]

        def win(t, carry):
            w = t * _NW + wid

            @pl.when(w < n_win)
            def _():
                base = w * _W
                pltpu.sync_copy(idx_hbm.at[pl.ds(base, _W)], idx_v)
                pltpu.async_copy(tbl_hbm.at[idx_v], rows_v, sem).wait()

                def grp(g, c2):
                    j = g * _L
                    j_idx = j + lane
                    for c in range(C):
                        vals = plsc.load_gather(rows_v, [j_idx, c_splat[c]])
                        trans_v[c, pl.ds(j, _L)] = vals
                    return c2

                lax.fori_loop(0, _W // _L, grp, 0)
                pltpu.sync_copy(trans_v, out_hbm.at[0].at[:, pl.ds(base, _W)])

            return carry

        lax.fori_loop(0, win_per_worker, win, 0)

    return gather_kernel


def kernel(input_features, aprs, level_deltas):
    B, C, n_in = input_features.shape
    n_out = aprs.shape[0]
    tbl_flat = _build_transpose(B * C, n_in)(input_features)
    tbl = tbl_flat.reshape(n_in, B * C)
    out = _build_gather(B * C, n_in, n_out)(tbl, aprs)
    return out.reshape(B, C, n_out)


# speedup vs baseline: 4.7013x; 4.5288x over previous
import functools

import jax
import jax.numpy as jnp
from jax import lax
from jax.experimental import pallas as pl
from jax.experimental.pallas import tpu as pltpu
from jax.experimental.pallas import tpu_sc as plsc

_NC = 2
_NS = 16
_NW = _NC * _NS
_L = 16

_W = 6400
_KA = 3200


def _build_transpose(C: int, n_in: int):
    n_full = n_in // _KA
    tail = n_in - n_full * _KA
    n_chunk = n_full + (1 if tail else 0)
    per_tile = -(-n_chunk // _NW)

    mesh = plsc.VectorSubcoreMesh(core_axis_name="c", subcore_axis_name="s")
    scratch = [
        pltpu.VMEM((C, _KA), jnp.float32),
        pltpu.VMEM((_KA * C,), jnp.float32),
    ]
    if tail:
        scratch += [
            pltpu.VMEM((C, tail), jnp.float32),
            pltpu.VMEM((tail * C,), jnp.float32),
        ]
    scratch.append(pltpu.SemaphoreType.DMA)

    @functools.partial(
        pl.kernel,
        mesh=mesh,
        out_type=jax.ShapeDtypeStruct((n_in * C,), jnp.float32),
        scratch_types=scratch,
        compiler_params=pltpu.CompilerParams(needs_layout_passes=False),
    )
    def transpose_kernel(feat_hbm, tbl_hbm, *rest):
        if tail:
            slab_v, rowb_v, slab2_v, rowb2_v, sem = rest
        else:
            slab_v, rowb_v, sem = rest
        core = lax.axis_index("c")
        sub = lax.axis_index("s")
        wid = sub * _NC + core
        lane_c = lax.iota(jnp.int32, _L) * C

        def do(slab, rowb, base, K):
            pltpu.sync_copy(feat_hbm.at[0].at[:, pl.ds(base, K)], slab)

            def grp(g, c2):
                j = g * _L
                for c in range(C):
                    vals = slab[c, pl.ds(j, _L)]
                    plsc.store_scatter(rowb, [j * C + lane_c + c], vals)
                return c2

            lax.fori_loop(0, K // _L, grp, 0)
            pltpu.sync_copy(rowb, tbl_hbm.at[pl.ds(base * C, K * C)])

        def chunk(t, carry):
            m = t * _NW + wid

            @pl.when(m < n_full)
            def _():
                do(slab_v, rowb_v, m * _KA, _KA)

            if tail:
                @pl.when(m == n_full)
                def _():
                    do(slab2_v, rowb2_v, n_full * _KA, tail)

            return carry

        lax.fori_loop(0, per_tile, chunk, 0)

    return transpose_kernel


def _build_gather(C: int, n_in: int, n_out: int):
    assert n_out % _W == 0
    n_win = n_out // _W
    win_per_worker = -(-n_win // _NW)

    mesh = plsc.VectorSubcoreMesh(core_axis_name="c", subcore_axis_name="s")

    @functools.partial(
        pl.kernel,
        mesh=mesh,
        out_type=jax.ShapeDtypeStruct((C * n_out,), jnp.float32),
        scratch_types=[
            pltpu.VMEM((_W,), jnp.int32),
            pltpu.VMEM((_W, C), jnp.float32),
            pltpu.VMEM((C, _W), jnp.float32),
            pltpu.SemaphoreType.DMA,
        ],
        compiler_params=pltpu.CompilerParams(
            use_tc_tiling_on_sc=False, needs_layout_passes=False
        ),
    )
    def gather_kernel(tbl_hbm, idx_hbm, out_hbm, idx_v, rows_v, trans_v, sem):
        core = lax.axis_index("c")
        sub = lax.axis_index("s")
        wid = sub * _NC + core
        lane = lax.iota(jnp.int32, _L)
        c_splat = [jnp.full((_L,), c, dtype=jnp.int32) for c in range(C)]

        def win(t, carry):
            w = t * _NW + wid

            @pl.when(w < n_win)
            def _():
                base = w * _W
                pltpu.sync_copy(idx_hbm.at[pl.ds(base, _W)], idx_v)
                pltpu.async_copy(tbl_hbm.at[idx_v], rows_v, sem).wait()

                def grp(g, c2):
                    j = g * _L
                    j_idx = j + lane
                    for c in range(C):
                        vals = plsc.load_gather(rows_v, [j_idx, c_splat[c]])
                        trans_v[c, pl.ds(j, _L)] = vals
                    return c2

                lax.fori_loop(0, _W // _L, grp, 0)
                for c in range(C):
                    pltpu.sync_copy(
                        trans_v.at[c], out_hbm.at[pl.ds(c * n_out + base, _W)]
                    )

            return carry

        lax.fori_loop(0, win_per_worker, win, 0)

    return gather_kernel


def _build_relayout(C: int, n_out: int):
    assert n_out % _W == 0
    n_win = n_out // _W
    win_per_worker = -(-n_win // _NW)

    mesh = plsc.VectorSubcoreMesh(core_axis_name="c", subcore_axis_name="s")

    @functools.partial(
        pl.kernel,
        mesh=mesh,
        out_type=jax.ShapeDtypeStruct((1, C, n_out), jnp.float32),
        scratch_types=[
            pltpu.VMEM((C, _W), jnp.float32),
            pltpu.SemaphoreType.DMA,
        ],
        compiler_params=pltpu.CompilerParams(needs_layout_passes=False),
    )
    def relayout_kernel(flat_hbm, out_hbm, slab_v, sem):
        core = lax.axis_index("c")
        sub = lax.axis_index("s")
        wid = sub * _NC + core

        def win(t, carry):
            w = t * _NW + wid

            @pl.when(w < n_win)
            def _():
                base = w * _W
                for c in range(C):
                    pltpu.sync_copy(
                        flat_hbm.at[pl.ds(c * n_out + base, _W)], slab_v.at[c]
                    )
                pltpu.sync_copy(slab_v, out_hbm.at[0].at[:, pl.ds(base, _W)])

            return carry

        lax.fori_loop(0, win_per_worker, win, 0)

    return relayout_kernel


def kernel(input_features, aprs, level_deltas):
    B, C, n_in = input_features.shape
    n_out = aprs.shape[0]
    tbl_flat = _build_transpose(B * C, n_in)(input_features)
    tbl = tbl_flat.reshape(n_in, B * C)
    out_flat = _build_gather(B * C, n_in, n_out)(tbl, aprs)
    out = _build_relayout(B * C, n_out)(out_flat)
    return out.reshape(B, C, n_out)


# speedup vs baseline: 5.6562x; 1.2031x over previous
import functools

import jax
import jax.numpy as jnp
from jax import lax
from jax.experimental import pallas as pl
from jax.experimental.pallas import tpu as pltpu
from jax.experimental.pallas import tpu_sc as plsc

_NC = 2
_NS = 16
_NW = _NC * _NS
_L = 16

_W = 6400
_KA = 3200


def _build_transpose(C: int, n_in: int):
    n_full = n_in // _KA
    tail = n_in - n_full * _KA
    n_chunk = n_full + (1 if tail else 0)
    per_tile = -(-n_chunk // _NW)

    mesh = plsc.VectorSubcoreMesh(core_axis_name="c", subcore_axis_name="s")
    scratch = [
        pltpu.VMEM((C, _KA), jnp.float32),
        pltpu.VMEM((_KA * C,), jnp.float32),
    ]
    if tail:
        scratch += [
            pltpu.VMEM((C, tail), jnp.float32),
            pltpu.VMEM((tail * C,), jnp.float32),
        ]
    scratch.append(pltpu.SemaphoreType.DMA)

    @functools.partial(
        pl.kernel,
        mesh=mesh,
        out_type=jax.ShapeDtypeStruct((n_in * C,), jnp.float32),
        scratch_types=scratch,
        compiler_params=pltpu.CompilerParams(needs_layout_passes=False),
    )
    def transpose_kernel(feat_hbm, tbl_hbm, *rest):
        if tail:
            slab_v, rowb_v, slab2_v, rowb2_v, sem = rest
        else:
            slab_v, rowb_v, sem = rest
        core = lax.axis_index("c")
        sub = lax.axis_index("s")
        wid = sub * _NC + core
        lane_c = lax.iota(jnp.int32, _L) * C

        def do(slab, rowb, base, K):
            pltpu.sync_copy(feat_hbm.at[0].at[:, pl.ds(base, K)], slab)

            def grp(g, c2):
                j = g * _L
                for c in range(C):
                    vals = slab[c, pl.ds(j, _L)]
                    plsc.store_scatter(rowb, [j * C + lane_c + c], vals)
                return c2

            lax.fori_loop(0, K // _L, grp, 0)
            pltpu.sync_copy(rowb, tbl_hbm.at[pl.ds(base * C, K * C)])

        def chunk(t, carry):
            m = t * _NW + wid

            @pl.when(m < n_full)
            def _():
                do(slab_v, rowb_v, m * _KA, _KA)

            if tail:
                @pl.when(m == n_full)
                def _():
                    do(slab2_v, rowb2_v, n_full * _KA, tail)

            return carry

        lax.fori_loop(0, per_tile, chunk, 0)

    return transpose_kernel


_WG = 4000


def _build_gather(C: int, n_in: int, n_out: int):
    assert n_out % _WG == 0
    n_win = n_out // _WG
    win_per_worker = -(-n_win // _NW)
    n_pair = -(-win_per_worker // 2)

    mesh = plsc.VectorSubcoreMesh(core_axis_name="c", subcore_axis_name="s")

    @functools.partial(
        pl.kernel,
        mesh=mesh,
        out_type=jax.ShapeDtypeStruct((C * n_out,), jnp.float32),
        scratch_types=[
            pltpu.VMEM((2, _WG), jnp.int32),
            pltpu.VMEM((2, _WG, C), jnp.float32),
            pltpu.VMEM((C, _WG), jnp.float32),
            pltpu.SemaphoreType.DMA,
            pltpu.SemaphoreType.DMA,
        ],
        compiler_params=pltpu.CompilerParams(
            use_tc_tiling_on_sc=False, needs_layout_passes=False
        ),
    )
    def gather_kernel(tbl_hbm, idx_hbm, out_hbm, idx_v, rows_v, trans_v, s0, s1):
        core = lax.axis_index("c")
        sub = lax.axis_index("s")
        wid = sub * _NC + core
        lane = lax.iota(jnp.int32, _L)
        c_splat = [jnp.full((_L,), c, dtype=jnp.int32) for c in range(C)]
        sems = (s0, s1)

        def fire(w, b):
            @pl.when(w < n_win)
            def _():
                base = w * _WG
                pltpu.sync_copy(idx_hbm.at[pl.ds(base, _WG)], idx_v.at[b])
                pltpu.async_copy(tbl_hbm.at[idx_v.at[b]], rows_v.at[b], sems[b])

        def drain_process(w, b):
            @pl.when(w < n_win)
            def _():
                pltpu.make_async_copy(
                    tbl_hbm.at[pl.ds(0, _WG), :], rows_v.at[b], sems[b]
                ).wait()
                base = w * _WG
                rows_b = rows_v.at[b]

                def grp(g, c2):
                    j = g * _L
                    j_idx = j + lane
                    for c in range(C):
                        vals = plsc.load_gather(rows_b, [j_idx, c_splat[c]])
                        trans_v[c, pl.ds(j, _L)] = vals
                    return c2

                lax.fori_loop(0, _WG // _L, grp, 0)
                for c in range(C):
                    pltpu.sync_copy(
                        trans_v.at[c], out_hbm.at[pl.ds(c * n_out + base, _WG)]
                    )

        def w_of(t):
            return t * _NW + wid

        fire(w_of(0), 0)

        def pair(u, carry):
            t0 = 2 * u
            fire(w_of(t0 + 1), 1)
            drain_process(w_of(t0), 0)
            fire(w_of(t0 + 2), 0)
            drain_process(w_of(t0 + 1), 1)
            return carry

        lax.fori_loop(0, n_pair, pair, 0)

    return gather_kernel


def _build_relayout(C: int, n_out: int):
    assert n_out % _W == 0
    n_win = n_out // _W
    win_per_worker = -(-n_win // _NW)

    mesh = plsc.VectorSubcoreMesh(core_axis_name="c", subcore_axis_name="s")

    @functools.partial(
        pl.kernel,
        mesh=mesh,
        out_type=jax.ShapeDtypeStruct((1, C, n_out), jnp.float32),
        scratch_types=[
            pltpu.VMEM((C, _W), jnp.float32),
            pltpu.SemaphoreType.DMA,
        ],
        compiler_params=pltpu.CompilerParams(needs_layout_passes=False),
    )
    def relayout_kernel(flat_hbm, out_hbm, slab_v, sem):
        core = lax.axis_index("c")
        sub = lax.axis_index("s")
        wid = sub * _NC + core

        def win(t, carry):
            w = t * _NW + wid

            @pl.when(w < n_win)
            def _():
                base = w * _W
                for c in range(C):
                    pltpu.sync_copy(
                        flat_hbm.at[pl.ds(c * n_out + base, _W)], slab_v.at[c]
                    )
                pltpu.sync_copy(slab_v, out_hbm.at[0].at[:, pl.ds(base, _W)])

            return carry

        lax.fori_loop(0, win_per_worker, win, 0)

    return relayout_kernel


def kernel(input_features, aprs, level_deltas):
    B, C, n_in = input_features.shape
    n_out = aprs.shape[0]
    tbl_flat = _build_transpose(B * C, n_in)(input_features)
    tbl = tbl_flat.reshape(n_in, B * C)
    out_flat = _build_gather(B * C, n_in, n_out)(tbl, aprs)
    out = _build_relayout(B * C, n_out)(out_flat)
    return out.reshape(B, C, n_out)
